# Initial kernel scaffold; baseline (speedup 1.0000x reference)
import jax
import jax.numpy as jnp
from jax.experimental import pallas as pl


def kernel(x, in_proj_w, conv_w, conv_b, W_q, W_k, W_v, W_alpha, b_alpha, W_beta, b_beta, out_proj_w):
    raise NotImplementedError("write your pallas kernel here")


# speedup vs baseline: 2.9556x; 2.9556x over previous
import jax
import jax.numpy as jnp
from jax import lax
from jax.experimental import pallas as pl
from jax.experimental.pallas import tpu as pltpu

DIM = 1024
D_INNER = 1024
HEAD_DIM = 64
NUM_HEADS = D_INNER // HEAD_DIM
D_CONV = 4
TB = 256
CHUNK = 64


def _proj_kernel(x_ref, xprev_ref, winT_ref, convw_ref, convb_ref,
                 wqT_ref, wkT_ref, wvT_ref, waT_ref, ba_ref, wbT_ref, bb_ref,
                 q_ref, k_ref, v_ref, a_ref, b_ref):
    i = pl.program_id(1)
    f32 = jnp.float32

    x_cur = x_ref[0]
    xp_cur = jnp.dot(x_cur, winT_ref[...], preferred_element_type=f32)
    xp_prev = jnp.dot(xprev_ref[0, 0], winT_ref[...], preferred_element_type=f32)
    xp_prev = jnp.where(i == 0, jnp.zeros_like(xp_prev), xp_prev)

    ext = jnp.concatenate([xp_prev, xp_cur], axis=0)
    xc = convw_ref[3:4, :] * ext[8:8 + TB]
    xc = xc + convw_ref[2:3, :] * ext[7:7 + TB]
    xc = xc + convw_ref[1:2, :] * ext[6:6 + TB]
    xc = xc + convw_ref[0:1, :] * ext[5:5 + TB]
    xc = xc + convb_ref[...]
    act = xc * jax.nn.sigmoid(xc)

    ridx = lax.broadcasted_iota(jnp.int32, (D_INNER, NUM_HEADS), 0)
    hidx = lax.broadcasted_iota(jnp.int32, (D_INNER, NUM_HEADS), 1)
    seg = (ridx // HEAD_DIM == hidx).astype(f32)

    def _norm(y):
        ss = jnp.dot(y * y, seg, preferred_element_type=f32)
        inv = 1.0 / jnp.maximum(jnp.sqrt(ss), 1e-12)
        inv_full = lax.dot_general(inv, seg, (((1,), (1,)), ((), ())),
                                   preferred_element_type=f32)
        return y * inv_full

    q_ref[0] = _norm(jnp.dot(act, wqT_ref[...], preferred_element_type=f32))
    k_ref[0] = _norm(jnp.dot(act, wkT_ref[...], preferred_element_type=f32))
    v_ref[0] = jnp.dot(act, wvT_ref[...], preferred_element_type=f32)

    asig = jax.nn.sigmoid(jnp.dot(act, waT_ref[...], preferred_element_type=f32)
                          + ba_ref[...])
    bsig = jax.nn.sigmoid(jnp.dot(act, wbT_ref[...], preferred_element_type=f32)
                          + bb_ref[...])
    a_ref[0] = jnp.dot(asig, seg, preferred_element_type=f32) * (1.0 / HEAD_DIM)
    b_ref[0] = jnp.dot(bsig, seg, preferred_element_type=f32) * (1.0 / HEAD_DIM)


def _delta_kernel(q_ref, k_ref, v_ref, a_ref, b_ref, woutT_ref,
                  out_ref, sfin_ref, z_ref):
    j = pl.program_id(1)
    f32 = jnp.float32
    C = CHUNK
    NC = 2048 // CHUNK

    @pl.when(j == 0)
    def _init():
        z_ref[...] = jnp.zeros_like(z_ref)

    qb = q_ref[0]
    kb = k_ref[0]
    vb = v_ref[0]
    al = jnp.maximum(a_ref[0], 1e-30)
    be = b_ref[0]

    t_i = lax.broadcasted_iota(jnp.int32, (C, C), 0)
    s_i = lax.broadcasted_iota(jnp.int32, (C, C), 1)
    tril_incl = (t_i >= s_i).astype(f32)
    ell = jnp.dot(tril_incl, jnp.log(al), preferred_element_type=f32)
    eyeC = (t_i == s_i).astype(f32)
    ellT = lax.dot_general(ell, eyeC, (((0,), (0,)), ((), ())),
                           preferred_element_type=f32)

    zall = z_ref[...]
    o_parts = []
    z_parts = []
    for h in range(NUM_HEADS):
        sl = slice(h * HEAD_DIM, (h + 1) * HEAD_DIM)
        Q = qb[:, sl]
        K = kb[:, sl]
        V = vb[:, sl]
        lc = ell[:, h:h + 1]
        lr = ellT[h:h + 1, :]
        D = jnp.exp(lc - lr)
        bc = be[:, h:h + 1]
        gam = jnp.exp(lc)

        Skk = lax.dot_general(K, K, (((1,), (1,)), ((), ())),
                              preferred_element_type=f32)
        Sqk = lax.dot_general(Q, K, (((1,), (1,)), ((), ())),
                              preferred_element_type=f32)
        G = jnp.where(t_i > s_i, Skk * D, 0.0) * bc
        A = jnp.where(t_i >= s_i, Sqk * D, 0.0)

        Z = zall[:, sl]
        R = (V - jnp.dot(K * gam, Z, preferred_element_type=f32)) * bc
        U = R - jnp.dot(G, R, preferred_element_type=f32)
        Gp = G
        for _ in range(5):
            Gp = jnp.dot(Gp, Gp, preferred_element_type=f32)
            U = U + jnp.dot(Gp, U, preferred_element_type=f32)

        O = (jnp.dot(A, U, preferred_element_type=f32)
             + jnp.dot(Q * gam, Z, preferred_element_type=f32))
        o_parts.append(O)

        gC = jnp.exp(ell[C - 1:C, h:h + 1])
        Ud = U * jnp.exp(ell[C - 1:C, h:h + 1] - lc)
        z_parts.append(Z * gC + lax.dot_general(K, Ud, (((0,), (0,)), ((), ())),
                                                preferred_element_type=f32))

    z_ref[...] = jnp.concatenate(z_parts, axis=1)
    o_full = jnp.concatenate(o_parts, axis=1)
    out_ref[0] = jnp.dot(o_full, woutT_ref[...], preferred_element_type=f32)

    @pl.when(j == NC - 1)
    def _fin():
        for h in range(NUM_HEADS):
            sfin_ref[0, h] = z_ref[:, h * HEAD_DIM:(h + 1) * HEAD_DIM].T


def kernel(x, in_proj_w, conv_w, conv_b, W_q, W_k, W_v, W_alpha, b_alpha,
           W_beta, b_beta, out_proj_w):
    Bsz, T, _ = x.shape
    f32 = jnp.float32
    winT = in_proj_w.T
    convw2 = conv_w[:, 0, :].T
    convb2 = conv_b[None, :]
    wqT, wkT, wvT = W_q.T, W_k.T, W_v.T
    waT, wbT = W_alpha.T, W_beta.T
    ba2, bb2 = b_alpha[None, :], b_beta[None, :]
    woutT = out_proj_w.T
    xh = x.reshape(Bsz, T // 8, 8, DIM)

    nblk = T // TB
    full = lambda b, i: (0, 0)
    q, k, v, al, be = pl.pallas_call(
        _proj_kernel,
        grid=(Bsz, nblk),
        in_specs=[
            pl.BlockSpec((1, TB, DIM), lambda b, i: (b, i, 0)),
            pl.BlockSpec((1, 1, 8, DIM),
                         lambda b, i: (b, jnp.maximum(i * (TB // 8) - 1, 0), 0, 0)),
            pl.BlockSpec((DIM, D_INNER), full),
            pl.BlockSpec((D_CONV, D_INNER), full),
            pl.BlockSpec((1, D_INNER), full),
            pl.BlockSpec((DIM, D_INNER), full),
            pl.BlockSpec((DIM, D_INNER), full),
            pl.BlockSpec((DIM, D_INNER), full),
            pl.BlockSpec((DIM, D_INNER), full),
            pl.BlockSpec((1, D_INNER), full),
            pl.BlockSpec((DIM, D_INNER), full),
            pl.BlockSpec((1, D_INNER), full),
        ],
        out_specs=[
            pl.BlockSpec((1, TB, D_INNER), lambda b, i: (b, i, 0)),
            pl.BlockSpec((1, TB, D_INNER), lambda b, i: (b, i, 0)),
            pl.BlockSpec((1, TB, D_INNER), lambda b, i: (b, i, 0)),
            pl.BlockSpec((1, TB, NUM_HEADS), lambda b, i: (b, i, 0)),
            pl.BlockSpec((1, TB, NUM_HEADS), lambda b, i: (b, i, 0)),
        ],
        out_shape=[
            jax.ShapeDtypeStruct((Bsz, T, D_INNER), f32),
            jax.ShapeDtypeStruct((Bsz, T, D_INNER), f32),
            jax.ShapeDtypeStruct((Bsz, T, D_INNER), f32),
            jax.ShapeDtypeStruct((Bsz, T, NUM_HEADS), f32),
            jax.ShapeDtypeStruct((Bsz, T, NUM_HEADS), f32),
        ],
        compiler_params=pltpu.CompilerParams(
            dimension_semantics=("parallel", "arbitrary"),
            vmem_limit_bytes=100 * 1024 * 1024,
        ),
        name="gdn_proj",
    )(x, xh, winT, convw2, convb2, wqT, wkT, wvT, waT, ba2, wbT, bb2)

    nchunk = T // CHUNK
    out, sfin = pl.pallas_call(
        _delta_kernel,
        grid=(Bsz, nchunk),
        in_specs=[
            pl.BlockSpec((1, CHUNK, D_INNER), lambda b, j: (b, j, 0)),
            pl.BlockSpec((1, CHUNK, D_INNER), lambda b, j: (b, j, 0)),
            pl.BlockSpec((1, CHUNK, D_INNER), lambda b, j: (b, j, 0)),
            pl.BlockSpec((1, CHUNK, NUM_HEADS), lambda b, j: (b, j, 0)),
            pl.BlockSpec((1, CHUNK, NUM_HEADS), lambda b, j: (b, j, 0)),
            pl.BlockSpec((D_INNER, DIM), lambda b, j: (0, 0)),
        ],
        out_specs=[
            pl.BlockSpec((1, CHUNK, DIM), lambda b, j: (b, j, 0)),
            pl.BlockSpec((1, NUM_HEADS, HEAD_DIM, HEAD_DIM),
                         lambda b, j: (b, 0, 0, 0)),
        ],
        out_shape=[
            jax.ShapeDtypeStruct((Bsz, T, DIM), f32),
            jax.ShapeDtypeStruct((Bsz, NUM_HEADS, HEAD_DIM, HEAD_DIM), f32),
        ],
        scratch_shapes=[pltpu.VMEM((HEAD_DIM, D_INNER), f32)],
        compiler_params=pltpu.CompilerParams(
            dimension_semantics=("parallel", "arbitrary"),
            vmem_limit_bytes=100 * 1024 * 1024,
        ),
        name="gdn_delta",
    )(q, k, v, al, be, woutT)

    return out, sfin
